# baseline (speedup 1.0000x reference)
import jax
import jax.numpy as jnp
from jax import lax
from jax.experimental import pallas as pl
from jax.experimental.pallas import tpu as pltpu
from jax.experimental.pallas import tpu_sc as plsc

_N = 10000
_D = 128
_NP = 10240
_BR = 256
_NPB = _NP // _BR
_NC = 2
_NS = 16
_NW = _NC * _NS
_CH = 128
_PH = 16
_RT = 80
_NPH = _RT // _PH
_EP = _NW * _RT * _CH
_TR = _NP // _NS


def _dot_t(a, w):
    return lax.dot_general(a, w, (((1,), (1,)), ((), ())),
                           preferred_element_type=jnp.float32)


def _lr(v):
    return jnp.where(v >= 0, v, 0.01 * v)


def _k1_body(mf_ref, f_ref, win_ref, bin_ref, x_ref, m_ref):
    mf = mf_ref[...]
    w = win_ref[...]
    b = bin_ref[...]
    x_ref[...] = _lr(_dot_t(mf, w) + b)
    m_ref[...] = _lr(_dot_t(f_ref[...] - mf, w) + b)


def _mlp_in(mf, f, win, binp):
    blk = pl.BlockSpec((_BR, _D), lambda i: (i, 0))
    full = pl.BlockSpec((_D, _D), lambda i: (0, 0))
    vec = pl.BlockSpec((1, _D), lambda i: (0, 0))
    return pl.pallas_call(
        _k1_body,
        grid=(_NPB,),
        in_specs=[blk, blk, full, vec],
        out_specs=[blk, blk],
        out_shape=[jax.ShapeDtypeStruct((_NP, _D), jnp.float32)] * 2,
    )(mf, f, win, binp)


def _mean_from_parts(a_ref, d0_ref, d1_ref):
    a = a_ref[0] + a_ref[1]
    d = jnp.clip(d0_ref[0, 0] + d1_ref[0, 0], 1.0, None)
    return a * (1.0 / d)[:, None]


def _k2_body(a_ref, d0_ref, d1_ref, x_ref, wl_ref, bl_ref, wr_ref, o_ref):
    mean = _mean_from_parts(a_ref, d0_ref, d1_ref)
    o_ref[...] = (_dot_t(mean, wl_ref[...]) + bl_ref[...]
                  + _dot_t(x_ref[...], wr_ref[...]))


def _sage_linear(aggp, d0, d1, x, wl, blp, wr):
    blk = pl.BlockSpec((_BR, _D), lambda i: (i, 0))
    pblk = pl.BlockSpec((_NC, _BR, _D), lambda i: (0, i, 0))
    dblk = pl.BlockSpec((1, 1, _BR), lambda i: (i, 0, 0))
    full = pl.BlockSpec((_D, _D), lambda i: (0, 0))
    vec = pl.BlockSpec((1, _D), lambda i: (0, 0))
    return pl.pallas_call(
        _k2_body,
        grid=(_NPB,),
        in_specs=[pblk, dblk, dblk, blk, full, vec, full],
        out_specs=blk,
        out_shape=jax.ShapeDtypeStruct((_NP, _D), jnp.float32),
    )(aggp, d0, d1, x, wl, blp, wr)


def _k3_body(a_ref, d0_ref, d1_ref, x2_ref, m_ref, wl_ref, bl_ref, wr_ref,
             wo_ref, bo_ref, o_ref):
    mean = _mean_from_parts(a_ref, d0_ref, d1_ref)
    x3 = (_dot_t(mean, wl_ref[...]) + bl_ref[...]
          + _dot_t(x2_ref[...], wr_ref[...]))
    o_ref[...] = ((_dot_t(x3, wo_ref[...]) + bo_ref[...])
                  * (_dot_t(m_ref[...], wo_ref[...]) + bo_ref[...]))


def _final(aggp, d0, d1, x2, m, wl, blp, wr, wop, bop):
    blk = pl.BlockSpec((_BR, _D), lambda i: (i, 0))
    pblk = pl.BlockSpec((_NC, _BR, _D), lambda i: (0, i, 0))
    dblk = pl.BlockSpec((1, 1, _BR), lambda i: (i, 0, 0))
    full = pl.BlockSpec((_D, _D), lambda i: (0, 0))
    vec = pl.BlockSpec((1, _D), lambda i: (0, 0))
    return pl.pallas_call(
        _k3_body,
        grid=(_NPB,),
        in_specs=[pblk, dblk, dblk, blk, blk, full, vec, full, full, vec],
        out_specs=blk,
        out_shape=jax.ShapeDtypeStruct((_NP, _D), jnp.float32),
    )(aggp, d0, d1, x2, m, wl, blp, wr, wop, bop)


_NPAIR = _PH // 2


def _make_agg(with_deg):
    outs = [jax.ShapeDtypeStruct((_NC, _NP, _D), jnp.float32)]
    scratch = [
        pltpu.VMEM((2, _PH, _CH), jnp.int32),
        pltpu.VMEM((2, _PH, _CH), jnp.int32),
        pltpu.VMEM((2, _CH, _D), jnp.float32),
        pltpu.VMEM_SHARED((_NP, _D), jnp.float32),
        pltpu.SemaphoreType.DMA,
        pltpu.SemaphoreType.DMA,
        pltpu.SemaphoreType.DMA,
        pltpu.SemaphoreType.DMA,
    ]
    if with_deg:
        outs.append(jax.ShapeDtypeStruct((_NC, _NP), jnp.float32))
        scratch += [
            pltpu.VMEM((_CH,), jnp.float32),
            pltpu.VMEM_SHARED((_NP,), jnp.float32),
            pltpu.SemaphoreType.DMA,
        ]

    def body(x_hbm, src_hbm, dst_hbm, z2_hbm, z1_hbm, one_hbm, *rest):
        if with_deg:
            (acc_out, deg_out, sidx, didx, rows, acc,
             gsem, ssem0, ssem1, isem, ones_v, deg, dsem) = rest
        else:
            (acc_out, sidx, didx, rows, acc, gsem, ssem0, ssem1, isem) = rest
        cid = lax.axis_index("c")
        sid = lax.axis_index("s")
        base = sid * _TR

        pltpu.sync_copy(z2_hbm, rows.at[0])
        for q in range(_TR // _CH):
            pltpu.sync_copy(rows.at[0], acc.at[pl.ds(base + q * _CH, _CH)])
        if with_deg:
            pltpu.sync_copy(z1_hbm, deg.at[pl.ds(base, _TR)])
            pltpu.sync_copy(one_hbm, ones_v)
        plsc.subcore_barrier()

        def wait_gather(b):
            pltpu.make_async_copy(x_hbm.at[sidx.at[0, 0]], rows.at[b],
                                  gsem).wait()

        def wait_scat(b, sem):
            pltpu.make_async_copy(rows.at[b], acc.at[didx.at[0, 0]],
                                  sem).wait()

        def wait_deg():
            pltpu.make_async_copy(ones_v, deg.at[didx.at[0, 0]], dsem).wait()

        wid = cid * _NS + sid
        wrow = wid * _RT

        def idx_load(h, p):
            pltpu.async_copy(src_hbm.at[pl.ds(wrow + h * _PH, _PH)],
                             sidx.at[p], isem)
            pltpu.async_copy(dst_hbm.at[pl.ds(wrow + h * _PH, _PH)],
                             didx.at[p], isem)

        def idx_wait():
            pltpu.make_async_copy(src_hbm.at[pl.ds(wrow, _PH)],
                                  sidx.at[0], isem).wait()
            pltpu.make_async_copy(dst_hbm.at[pl.ds(wrow, _PH)],
                                  didx.at[0], isem).wait()

        def gath(sv, c, b):
            h0 = _CH // 2
            pltpu.async_copy(x_hbm.at[sv.at[c].at[pl.ds(0, h0)]],
                             rows.at[b].at[pl.ds(0, h0)], gsem)
            pltpu.async_copy(x_hbm.at[sv.at[c].at[pl.ds(h0, h0)]],
                             rows.at[b].at[pl.ds(h0, h0)], gsem)

        def phase(p):
            sv = sidx.at[p]
            dv = didx.at[p]
            gath(sv, 0, 0)

            def pair(g, carry):
                c0 = 2 * g
                c1 = c0 + 1
                wait_gather(0)
                pltpu.async_copy(rows.at[0], acc.at[dv.at[c0]],
                                 ssem0, add=True)
                if with_deg:
                    pltpu.async_copy(ones_v, deg.at[dv.at[c0]],
                                     dsem, add=True)

                @pl.when(g > 0)
                def _():
                    wait_scat(1, ssem1)
                    if with_deg:
                        wait_deg()

                gath(sv, c1, 1)
                wait_gather(1)
                pltpu.async_copy(rows.at[1], acc.at[dv.at[c1]],
                                 ssem1, add=True)
                if with_deg:
                    pltpu.async_copy(ones_v, deg.at[dv.at[c1]],
                                     dsem, add=True)
                wait_scat(0, ssem0)
                if with_deg:
                    wait_deg()

                @pl.when(g < _NPAIR - 1)
                def _():
                    gath(sv, c0 + 2, 0)
                return carry

            lax.fori_loop(0, _NPAIR, pair, 0)
            wait_scat(1, ssem1)
            if with_deg:
                wait_deg()

        idx_load(0, 0)
        idx_wait()
        for h in range(_NPH):
            if h + 1 < _NPH:
                idx_load(h + 1, (h + 1) % 2)
            phase(h % 2)
            if h + 1 < _NPH:
                idx_wait()
        plsc.subcore_barrier()
        pltpu.sync_copy(acc.at[pl.ds(base, _TR)],
                        acc_out.at[cid].at[pl.ds(base, _TR)])
        if with_deg:
            pltpu.sync_copy(deg.at[pl.ds(base, _TR)],
                            deg_out.at[cid].at[pl.ds(base, _TR)])

    def _mesh():
        return plsc.VectorSubcoreMesh(core_axis_name="c", subcore_axis_name="s",
                                      num_cores=_NC, num_subcores=_NS)

    if with_deg:
        def run(x, src2d, dst2d, z2, z1, one):
            return pl.kernel(body, out_type=outs, mesh=_mesh(),
                             scratch_types=scratch)(x, src2d, dst2d, z2, z1, one)
    else:
        def run(x, src2d, dst2d, z2, z1, one):
            def body2(x_hbm, src_hbm, dst_hbm, z2_hbm, *rest):
                return body(x_hbm, src_hbm, dst_hbm, z2_hbm, None, None, *rest)
            return pl.kernel(body2, out_type=outs, mesh=_mesh(),
                             scratch_types=scratch)(x, src2d, dst2d, z2)
    return run


_agg_deg = _make_agg(True)
_agg_only = _make_agg(False)


def kernel(mask_feature, feature, edge_index, edge_type, W_in, b_in,
           W1l, b1l, W1r, W2l, b2l, W2r, Wout, bout):
    mf = jnp.pad(mask_feature, ((0, _NP - _N), (0, 0)))
    f = jnp.pad(feature, ((0, _NP - _N), (0, 0)))
    src = edge_index[0]
    dst = edge_index[1]
    pad_e = _EP - src.shape[0]
    pad_ar = jnp.arange(pad_e, dtype=jnp.int32)
    src2d = jnp.concatenate([src, pad_ar % _N]).reshape(_EP // _CH, _CH)
    dst2d = jnp.concatenate([dst, _N + pad_ar % (_NP - _N)]).reshape(_EP // _CH, _CH)
    z2 = jnp.zeros((_CH, _D), jnp.float32)
    z1 = jnp.zeros((_TR,), jnp.float32)
    one = jnp.ones((_CH,), jnp.float32)

    wop = jnp.pad(Wout, ((0, _D - Wout.shape[0]), (0, 0)))
    bop = jnp.pad(bout, (0, _D - bout.shape[0])).reshape(1, _D)

    x, m = _mlp_in(mf, f, W_in, b_in.reshape(1, _D))

    agg1, degp = _agg_deg(x, src2d, dst2d, z2, z1, one)
    d0 = degp[0].reshape(_NPB, 1, _BR)
    d1 = degp[1].reshape(_NPB, 1, _BR)

    x2 = _sage_linear(agg1, d0, d1, x, W1l, b1l.reshape(1, _D), W1r)

    agg2 = _agg_only(x2, src2d, dst2d, z2, z1, one)
    if isinstance(agg2, (list, tuple)):
        agg2 = agg2[0]

    out = _final(agg2, d0, d1, x2, m, W2l, b2l.reshape(1, _D), W2r, wop, bop)
    return out[:_N, :3]


# speedup vs baseline: 1.1399x; 1.1399x over previous
import jax
import jax.numpy as jnp
from jax import lax
from jax.experimental import pallas as pl
from jax.experimental.pallas import tpu as pltpu
from jax.experimental.pallas import tpu_sc as plsc

_N = 10000
_D = 128
_NP = 10240
_BR = 1024
_NPB = _NP // _BR
_NC = 2
_NS = 16
_NW = _NC * _NS
_CH = 128
_PH = 16
_RT = 80
_NPH = _RT // _PH
_EP = _NW * _RT * _CH
_TR = _NP // _NS


def _dot_t(a, w):
    return lax.dot_general(a, w, (((1,), (1,)), ((), ())),
                           preferred_element_type=jnp.float32)


def _lr(v):
    return jnp.where(v >= 0, v, 0.01 * v)


def _k1_body(mf_ref, f_ref, win_ref, bin_ref, x_ref, m_ref):
    mf = mf_ref[...]
    w = win_ref[...]
    b = bin_ref[...]
    x_ref[...] = _lr(_dot_t(mf, w) + b)
    m_ref[...] = _lr(_dot_t(f_ref[...] - mf, w) + b)


def _mlp_in(mf, f, win, binp):
    blk = pl.BlockSpec((_BR, _D), lambda i: (i, 0))
    full = pl.BlockSpec((_D, _D), lambda i: (0, 0))
    vec = pl.BlockSpec((1, _D), lambda i: (0, 0))
    return pl.pallas_call(
        _k1_body,
        grid=(_NPB,),
        in_specs=[blk, blk, full, vec],
        out_specs=[blk, blk],
        out_shape=[jax.ShapeDtypeStruct((_NP, _D), jnp.float32)] * 2,
    )(mf, f, win, binp)


def _mean_from_parts(a_ref, d0_ref, d1_ref):
    a = a_ref[0] + a_ref[1]
    d = jnp.clip(d0_ref[0, 0] + d1_ref[0, 0], 1.0, None)
    return a * (1.0 / d)[:, None]


def _k2_body(a_ref, d0_ref, d1_ref, x_ref, wl_ref, bl_ref, wr_ref, o_ref):
    mean = _mean_from_parts(a_ref, d0_ref, d1_ref)
    o_ref[...] = (_dot_t(mean, wl_ref[...]) + bl_ref[...]
                  + _dot_t(x_ref[...], wr_ref[...]))


def _sage_linear(aggp, d0, d1, x, wl, blp, wr):
    blk = pl.BlockSpec((_BR, _D), lambda i: (i, 0))
    pblk = pl.BlockSpec((_NC, _BR, _D), lambda i: (0, i, 0))
    dblk = pl.BlockSpec((1, 1, _BR), lambda i: (i, 0, 0))
    full = pl.BlockSpec((_D, _D), lambda i: (0, 0))
    vec = pl.BlockSpec((1, _D), lambda i: (0, 0))
    return pl.pallas_call(
        _k2_body,
        grid=(_NPB,),
        in_specs=[pblk, dblk, dblk, blk, full, vec, full],
        out_specs=blk,
        out_shape=jax.ShapeDtypeStruct((_NP, _D), jnp.float32),
    )(aggp, d0, d1, x, wl, blp, wr)


def _k3_body(a_ref, d0_ref, d1_ref, x2_ref, m_ref, wl_ref, bl_ref, wr_ref,
             wo_ref, bo_ref, o_ref):
    mean = _mean_from_parts(a_ref, d0_ref, d1_ref)
    x3 = (_dot_t(mean, wl_ref[...]) + bl_ref[...]
          + _dot_t(x2_ref[...], wr_ref[...]))
    o_ref[...] = ((_dot_t(x3, wo_ref[...]) + bo_ref[...])
                  * (_dot_t(m_ref[...], wo_ref[...]) + bo_ref[...]))


def _final(aggp, d0, d1, x2, m, wl, blp, wr, wop, bop):
    blk = pl.BlockSpec((_BR, _D), lambda i: (i, 0))
    pblk = pl.BlockSpec((_NC, _BR, _D), lambda i: (0, i, 0))
    dblk = pl.BlockSpec((1, 1, _BR), lambda i: (i, 0, 0))
    full = pl.BlockSpec((_D, _D), lambda i: (0, 0))
    vec = pl.BlockSpec((1, _D), lambda i: (0, 0))
    return pl.pallas_call(
        _k3_body,
        grid=(_NPB,),
        in_specs=[pblk, dblk, dblk, blk, blk, full, vec, full, full, vec],
        out_specs=blk,
        out_shape=jax.ShapeDtypeStruct((_NP, _D), jnp.float32),
    )(aggp, d0, d1, x2, m, wl, blp, wr, wop, bop)


_NPAIR = _PH // 2


def _make_agg(with_deg):
    outs = [jax.ShapeDtypeStruct((_NC, _NP, _D), jnp.float32)]
    scratch = [
        pltpu.VMEM((2, _PH, _CH), jnp.int32),
        pltpu.VMEM((2, _PH, _CH), jnp.int32),
        pltpu.VMEM((2, _CH, _D), jnp.float32),
        pltpu.VMEM_SHARED((_NP, _D), jnp.float32),
        pltpu.SemaphoreType.DMA,
        pltpu.SemaphoreType.DMA,
        pltpu.SemaphoreType.DMA,
        pltpu.SemaphoreType.DMA,
    ]
    if with_deg:
        outs.append(jax.ShapeDtypeStruct((_NC, _NP), jnp.float32))
        scratch += [
            pltpu.VMEM((_CH,), jnp.float32),
            pltpu.VMEM_SHARED((_NP,), jnp.float32),
            pltpu.SemaphoreType.DMA,
        ]

    def body(x_hbm, src_hbm, dst_hbm, z2_hbm, z1_hbm, one_hbm, *rest):
        if with_deg:
            (acc_out, deg_out, sidx, didx, rows, acc,
             gsem, ssem0, ssem1, isem, ones_v, deg, dsem) = rest
        else:
            (acc_out, sidx, didx, rows, acc, gsem, ssem0, ssem1, isem) = rest
        cid = lax.axis_index("c")
        sid = lax.axis_index("s")
        base = sid * _TR

        pltpu.sync_copy(z2_hbm, rows.at[0])
        for q in range(_TR // _CH):
            pltpu.sync_copy(rows.at[0], acc.at[pl.ds(base + q * _CH, _CH)])
        if with_deg:
            pltpu.sync_copy(z1_hbm, deg.at[pl.ds(base, _TR)])
            pltpu.sync_copy(one_hbm, ones_v)
        plsc.subcore_barrier()

        def wait_gather(b):
            pltpu.make_async_copy(x_hbm.at[sidx.at[0, 0]], rows.at[b],
                                  gsem).wait()

        def wait_scat(b, sem):
            pltpu.make_async_copy(rows.at[b], acc.at[didx.at[0, 0]],
                                  sem).wait()

        def wait_deg():
            pltpu.make_async_copy(ones_v, deg.at[didx.at[0, 0]], dsem).wait()

        wid = cid * _NS + sid
        wrow = wid * _RT

        def idx_load(h, p):
            pltpu.async_copy(src_hbm.at[pl.ds(wrow + h * _PH, _PH)],
                             sidx.at[p], isem)
            pltpu.async_copy(dst_hbm.at[pl.ds(wrow + h * _PH, _PH)],
                             didx.at[p], isem)

        def idx_wait():
            pltpu.make_async_copy(src_hbm.at[pl.ds(wrow, _PH)],
                                  sidx.at[0], isem).wait()
            pltpu.make_async_copy(dst_hbm.at[pl.ds(wrow, _PH)],
                                  didx.at[0], isem).wait()

        def gath(sv, c, b):
            h0 = _CH // 2
            pltpu.async_copy(x_hbm.at[sv.at[c].at[pl.ds(0, h0)]],
                             rows.at[b].at[pl.ds(0, h0)], gsem)
            pltpu.async_copy(x_hbm.at[sv.at[c].at[pl.ds(h0, h0)]],
                             rows.at[b].at[pl.ds(h0, h0)], gsem)

        def phase(p):
            sv = sidx.at[p]
            dv = didx.at[p]
            gath(sv, 0, 0)

            def pair(g, carry):
                c0 = 2 * g
                c1 = c0 + 1
                wait_gather(0)
                pltpu.async_copy(rows.at[0], acc.at[dv.at[c0]],
                                 ssem0, add=True)
                if with_deg:
                    pltpu.async_copy(ones_v, deg.at[dv.at[c0]],
                                     dsem, add=True)

                @pl.when(g > 0)
                def _():
                    wait_scat(1, ssem1)
                    if with_deg:
                        wait_deg()

                gath(sv, c1, 1)
                wait_gather(1)
                pltpu.async_copy(rows.at[1], acc.at[dv.at[c1]],
                                 ssem1, add=True)
                if with_deg:
                    pltpu.async_copy(ones_v, deg.at[dv.at[c1]],
                                     dsem, add=True)
                wait_scat(0, ssem0)
                if with_deg:
                    wait_deg()

                @pl.when(g < _NPAIR - 1)
                def _():
                    gath(sv, c0 + 2, 0)
                return carry

            lax.fori_loop(0, _NPAIR, pair, 0)
            wait_scat(1, ssem1)
            if with_deg:
                wait_deg()

        idx_load(0, 0)
        idx_wait()
        for h in range(_NPH):
            if h + 1 < _NPH:
                idx_load(h + 1, (h + 1) % 2)
            phase(h % 2)
            if h + 1 < _NPH:
                idx_wait()
        plsc.subcore_barrier()
        pltpu.sync_copy(acc.at[pl.ds(base, _TR)],
                        acc_out.at[cid].at[pl.ds(base, _TR)])
        if with_deg:
            pltpu.sync_copy(deg.at[pl.ds(base, _TR)],
                            deg_out.at[cid].at[pl.ds(base, _TR)])

    def _mesh():
        return plsc.VectorSubcoreMesh(core_axis_name="c", subcore_axis_name="s",
                                      num_cores=_NC, num_subcores=_NS)

    if with_deg:
        def run(x, src2d, dst2d, z2, z1, one):
            return pl.kernel(body, out_type=outs, mesh=_mesh(),
                             scratch_types=scratch)(x, src2d, dst2d, z2, z1, one)
    else:
        def run(x, src2d, dst2d, z2, z1, one):
            def body2(x_hbm, src_hbm, dst_hbm, z2_hbm, *rest):
                return body(x_hbm, src_hbm, dst_hbm, z2_hbm, None, None, *rest)
            return pl.kernel(body2, out_type=outs, mesh=_mesh(),
                             scratch_types=scratch)(x, src2d, dst2d, z2)
    return run


_agg_deg = _make_agg(True)
_agg_only = _make_agg(False)


def kernel(mask_feature, feature, edge_index, edge_type, W_in, b_in,
           W1l, b1l, W1r, W2l, b2l, W2r, Wout, bout):
    mf = jnp.pad(mask_feature, ((0, _NP - _N), (0, 0)))
    f = jnp.pad(feature, ((0, _NP - _N), (0, 0)))
    src = edge_index[0]
    dst = edge_index[1]
    pad_e = _EP - src.shape[0]
    pad_ar = jnp.arange(pad_e, dtype=jnp.int32)
    src2d = jnp.concatenate([src, pad_ar % _N]).reshape(_EP // _CH, _CH)
    dst2d = jnp.concatenate([dst, _N + pad_ar % (_NP - _N)]).reshape(_EP // _CH, _CH)
    z2 = jnp.zeros((_CH, _D), jnp.float32)
    z1 = jnp.zeros((_TR,), jnp.float32)
    one = jnp.ones((_CH,), jnp.float32)

    wop = jnp.pad(Wout, ((0, _D - Wout.shape[0]), (0, 0)))
    bop = jnp.pad(bout, (0, _D - bout.shape[0])).reshape(1, _D)

    x, m = _mlp_in(mf, f, W_in, b_in.reshape(1, _D))

    agg1, degp = _agg_deg(x, src2d, dst2d, z2, z1, one)
    d0 = degp[0].reshape(_NPB, 1, _BR)
    d1 = degp[1].reshape(_NPB, 1, _BR)

    x2 = _sage_linear(agg1, d0, d1, x, W1l, b1l.reshape(1, _D), W1r)

    agg2 = _agg_only(x2, src2d, dst2d, z2, z1, one)
    if isinstance(agg2, (list, tuple)):
        agg2 = agg2[0]

    out = _final(agg2, d0, d1, x2, m, W2l, b2l.reshape(1, _D), W2r, wop, bop)
    return out[:_N, :3]


# speedup vs baseline: 1.1632x; 1.0205x over previous
import jax
import jax.numpy as jnp
from jax import lax
from jax.experimental import pallas as pl
from jax.experimental.pallas import tpu as pltpu
from jax.experimental.pallas import tpu_sc as plsc

_N = 10000
_D = 128
_NP = 10240
_BR = 2048
_NPB = _NP // _BR
_NC = 2
_NS = 16
_NW = _NC * _NS
_CH = 128
_PH = 16
_RT = 80
_NPH = _RT // _PH
_EP = _NW * _RT * _CH
_TR = _NP // _NS


def _dot_t(a, w):
    return lax.dot_general(a, w, (((1,), (1,)), ((), ())),
                           preferred_element_type=jnp.float32)


def _lr(v):
    return jnp.where(v >= 0, v, 0.01 * v)


def _k1_body(mf_ref, f_ref, win_ref, bin_ref, x_ref, m_ref):
    mf = mf_ref[...]
    w = win_ref[...]
    b = bin_ref[...]
    x_ref[...] = _lr(_dot_t(mf, w) + b)
    m_ref[...] = _lr(_dot_t(f_ref[...] - mf, w) + b)


def _mlp_in(mf, f, win, binp):
    blk = pl.BlockSpec((_BR, _D), lambda i: (i, 0))
    full = pl.BlockSpec((_D, _D), lambda i: (0, 0))
    vec = pl.BlockSpec((1, _D), lambda i: (0, 0))
    return pl.pallas_call(
        _k1_body,
        grid=(_NPB,),
        in_specs=[blk, blk, full, vec],
        out_specs=[blk, blk],
        out_shape=[jax.ShapeDtypeStruct((_NP, _D), jnp.float32)] * 2,
    )(mf, f, win, binp)


def _mean_from_parts(a_ref, d0_ref, d1_ref):
    a = a_ref[0] + a_ref[1]
    d = jnp.clip(d0_ref[0, 0] + d1_ref[0, 0], 1.0, None)
    return a * (1.0 / d)[:, None]


def _k2_body(a_ref, d0_ref, d1_ref, x_ref, wl_ref, bl_ref, wr_ref, o_ref):
    mean = _mean_from_parts(a_ref, d0_ref, d1_ref)
    o_ref[...] = (_dot_t(mean, wl_ref[...]) + bl_ref[...]
                  + _dot_t(x_ref[...], wr_ref[...]))


def _sage_linear(aggp, d0, d1, x, wl, blp, wr):
    blk = pl.BlockSpec((_BR, _D), lambda i: (i, 0))
    pblk = pl.BlockSpec((_NC, _BR, _D), lambda i: (0, i, 0))
    dblk = pl.BlockSpec((1, 1, _BR), lambda i: (i, 0, 0))
    full = pl.BlockSpec((_D, _D), lambda i: (0, 0))
    vec = pl.BlockSpec((1, _D), lambda i: (0, 0))
    return pl.pallas_call(
        _k2_body,
        grid=(_NPB,),
        in_specs=[pblk, dblk, dblk, blk, full, vec, full],
        out_specs=blk,
        out_shape=jax.ShapeDtypeStruct((_NP, _D), jnp.float32),
    )(aggp, d0, d1, x, wl, blp, wr)


def _k3_body(a_ref, d0_ref, d1_ref, x2_ref, m_ref, wl_ref, bl_ref, wr_ref,
             wo_ref, bo_ref, o_ref):
    mean = _mean_from_parts(a_ref, d0_ref, d1_ref)
    x3 = (_dot_t(mean, wl_ref[...]) + bl_ref[...]
          + _dot_t(x2_ref[...], wr_ref[...]))
    o_ref[...] = ((_dot_t(x3, wo_ref[...]) + bo_ref[...])
                  * (_dot_t(m_ref[...], wo_ref[...]) + bo_ref[...]))


def _final(aggp, d0, d1, x2, m, wl, blp, wr, wop, bop):
    blk = pl.BlockSpec((_BR, _D), lambda i: (i, 0))
    pblk = pl.BlockSpec((_NC, _BR, _D), lambda i: (0, i, 0))
    dblk = pl.BlockSpec((1, 1, _BR), lambda i: (i, 0, 0))
    full = pl.BlockSpec((_D, _D), lambda i: (0, 0))
    vec = pl.BlockSpec((1, _D), lambda i: (0, 0))
    return pl.pallas_call(
        _k3_body,
        grid=(_NPB,),
        in_specs=[pblk, dblk, dblk, blk, blk, full, vec, full, full, vec],
        out_specs=blk,
        out_shape=jax.ShapeDtypeStruct((_NP, _D), jnp.float32),
    )(aggp, d0, d1, x2, m, wl, blp, wr, wop, bop)


_NPAIR = _PH // 2


def _make_agg(with_deg):
    outs = [jax.ShapeDtypeStruct((_NC, _NP, _D), jnp.float32)]
    scratch = [
        pltpu.VMEM((2, _PH, _CH), jnp.int32),
        pltpu.VMEM((2, _PH, _CH), jnp.int32),
        pltpu.VMEM((2, _CH, _D), jnp.float32),
        pltpu.VMEM_SHARED((_NP, _D), jnp.float32),
        pltpu.SemaphoreType.DMA,
        pltpu.SemaphoreType.DMA,
        pltpu.SemaphoreType.DMA,
        pltpu.SemaphoreType.DMA,
    ]
    if with_deg:
        outs.append(jax.ShapeDtypeStruct((_NC, _NP), jnp.float32))
        scratch += [
            pltpu.VMEM((_CH,), jnp.float32),
            pltpu.VMEM_SHARED((_NP,), jnp.float32),
            pltpu.SemaphoreType.DMA,
        ]

    def body(x_hbm, src_hbm, dst_hbm, z2_hbm, z1_hbm, one_hbm, *rest):
        if with_deg:
            (acc_out, deg_out, sidx, didx, rows, acc,
             gsem, ssem0, ssem1, isem, ones_v, deg, dsem) = rest
        else:
            (acc_out, sidx, didx, rows, acc, gsem, ssem0, ssem1, isem) = rest
        cid = lax.axis_index("c")
        sid = lax.axis_index("s")
        base = sid * _TR

        pltpu.sync_copy(z2_hbm, rows.at[0])
        for q in range(_TR // _CH):
            pltpu.sync_copy(rows.at[0], acc.at[pl.ds(base + q * _CH, _CH)])
        if with_deg:
            pltpu.sync_copy(z1_hbm, deg.at[pl.ds(base, _TR)])
            pltpu.sync_copy(one_hbm, ones_v)
        plsc.subcore_barrier()

        def wait_gather(b):
            pltpu.make_async_copy(x_hbm.at[sidx.at[0, 0]], rows.at[b],
                                  gsem).wait()

        def wait_scat(b, sem):
            pltpu.make_async_copy(rows.at[b], acc.at[didx.at[0, 0]],
                                  sem).wait()

        def wait_deg():
            pltpu.make_async_copy(ones_v, deg.at[didx.at[0, 0]], dsem).wait()

        wid = cid * _NS + sid
        wrow = wid * _RT

        def idx_load(h, p):
            pltpu.async_copy(src_hbm.at[pl.ds(wrow + h * _PH, _PH)],
                             sidx.at[p], isem)
            pltpu.async_copy(dst_hbm.at[pl.ds(wrow + h * _PH, _PH)],
                             didx.at[p], isem)

        def idx_wait():
            pltpu.make_async_copy(src_hbm.at[pl.ds(wrow, _PH)],
                                  sidx.at[0], isem).wait()
            pltpu.make_async_copy(dst_hbm.at[pl.ds(wrow, _PH)],
                                  didx.at[0], isem).wait()

        def gath(sv, c, b):
            h0 = _CH // 2
            pltpu.async_copy(x_hbm.at[sv.at[c].at[pl.ds(0, h0)]],
                             rows.at[b].at[pl.ds(0, h0)], gsem)
            pltpu.async_copy(x_hbm.at[sv.at[c].at[pl.ds(h0, h0)]],
                             rows.at[b].at[pl.ds(h0, h0)], gsem)

        def phase(p):
            sv = sidx.at[p]
            dv = didx.at[p]
            gath(sv, 0, 0)

            def pair(g, carry):
                c0 = 2 * g
                c1 = c0 + 1
                wait_gather(0)
                pltpu.async_copy(rows.at[0], acc.at[dv.at[c0]],
                                 ssem0, add=True)
                if with_deg:
                    pltpu.async_copy(ones_v, deg.at[dv.at[c0]],
                                     dsem, add=True)

                @pl.when(g > 0)
                def _():
                    wait_scat(1, ssem1)
                    if with_deg:
                        wait_deg()

                gath(sv, c1, 1)
                wait_gather(1)
                pltpu.async_copy(rows.at[1], acc.at[dv.at[c1]],
                                 ssem1, add=True)
                if with_deg:
                    pltpu.async_copy(ones_v, deg.at[dv.at[c1]],
                                     dsem, add=True)
                wait_scat(0, ssem0)
                if with_deg:
                    wait_deg()

                @pl.when(g < _NPAIR - 1)
                def _():
                    gath(sv, c0 + 2, 0)
                return carry

            lax.fori_loop(0, _NPAIR, pair, 0)
            wait_scat(1, ssem1)
            if with_deg:
                wait_deg()

        idx_load(0, 0)
        idx_wait()
        for h in range(_NPH):
            if h + 1 < _NPH:
                idx_load(h + 1, (h + 1) % 2)
            phase(h % 2)
            if h + 1 < _NPH:
                idx_wait()
        plsc.subcore_barrier()
        pltpu.sync_copy(acc.at[pl.ds(base, _TR)],
                        acc_out.at[cid].at[pl.ds(base, _TR)])
        if with_deg:
            pltpu.sync_copy(deg.at[pl.ds(base, _TR)],
                            deg_out.at[cid].at[pl.ds(base, _TR)])

    def _mesh():
        return plsc.VectorSubcoreMesh(core_axis_name="c", subcore_axis_name="s",
                                      num_cores=_NC, num_subcores=_NS)

    if with_deg:
        def run(x, src2d, dst2d, z2, z1, one):
            return pl.kernel(body, out_type=outs, mesh=_mesh(),
                             scratch_types=scratch)(x, src2d, dst2d, z2, z1, one)
    else:
        def run(x, src2d, dst2d, z2, z1, one):
            def body2(x_hbm, src_hbm, dst_hbm, z2_hbm, *rest):
                return body(x_hbm, src_hbm, dst_hbm, z2_hbm, None, None, *rest)
            return pl.kernel(body2, out_type=outs, mesh=_mesh(),
                             scratch_types=scratch)(x, src2d, dst2d, z2)
    return run


_agg_deg = _make_agg(True)
_agg_only = _make_agg(False)


def kernel(mask_feature, feature, edge_index, edge_type, W_in, b_in,
           W1l, b1l, W1r, W2l, b2l, W2r, Wout, bout):
    mf = jnp.pad(mask_feature, ((0, _NP - _N), (0, 0)))
    f = jnp.pad(feature, ((0, _NP - _N), (0, 0)))
    src = edge_index[0]
    dst = edge_index[1]
    pad_e = _EP - src.shape[0]
    pad_ar = jnp.arange(pad_e, dtype=jnp.int32)
    src2d = jnp.concatenate([src, pad_ar % _N]).reshape(_EP // _CH, _CH)
    dst2d = jnp.concatenate([dst, _N + pad_ar % (_NP - _N)]).reshape(_EP // _CH, _CH)
    z2 = jnp.zeros((_CH, _D), jnp.float32)
    z1 = jnp.zeros((_TR,), jnp.float32)
    one = jnp.ones((_CH,), jnp.float32)

    wop = jnp.pad(Wout, ((0, _D - Wout.shape[0]), (0, 0)))
    bop = jnp.pad(bout, (0, _D - bout.shape[0])).reshape(1, _D)

    x, m = _mlp_in(mf, f, W_in, b_in.reshape(1, _D))

    agg1, degp = _agg_deg(x, src2d, dst2d, z2, z1, one)
    d0 = degp[0].reshape(_NPB, 1, _BR)
    d1 = degp[1].reshape(_NPB, 1, _BR)

    x2 = _sage_linear(agg1, d0, d1, x, W1l, b1l.reshape(1, _D), W1r)

    agg2 = _agg_only(x2, src2d, dst2d, z2, z1, one)
    if isinstance(agg2, (list, tuple)):
        agg2 = agg2[0]

    out = _final(agg2, d0, d1, x2, m, W2l, b2l.reshape(1, _D), W2r, wop, bop)
    return out[:_N, :3]
